# baseline (speedup 1.0000x reference)
import functools

import jax
import jax.numpy as jnp
from jax import lax
from jax.experimental import pallas as pl
from jax.experimental.pallas import tpu as pltpu
from jax.experimental.pallas import tpu_sc as plsc

N = 10000
E = 320000
D = 128

NC = 2
NS = 16
L = 16

E2 = E // NC
CH = 1280
NCHUNK = E2 // CH
ROWS = E // CH
GROUPS = CH // L
GU = 4
NB = 2

N_PAD = 10240


def _sc_partial(edge_t, il3, zeros8):
    mesh = plsc.VectorSubcoreMesh(core_axis_name="c", subcore_axis_name="s")

    @functools.partial(
        pl.kernel,
        out_type=jax.ShapeDtypeStruct((NC, D, N_PAD), jnp.float32),
        mesh=mesh,
        scratch_types=[
            pltpu.VMEM((NB, 2, CH), jnp.int32),
            pltpu.VMEM((NB, 8, CH), jnp.float32),
            pltpu.VMEM((8, N_PAD), jnp.float32),
            pltpu.SemaphoreType.DMA((NB,)),
            pltpu.SemaphoreType.DMA((NB,)),
        ],
        compiler_params=pltpu.CompilerParams(needs_layout_passes=False),
    )
    def k(edge_hbm, il_hbm, zeros_hbm, part_hbm,
          il_v, stage, acc, isem, rsem):
        c = lax.axis_index("c")
        s = lax.axis_index("s")
        is_tile0 = s == 0
        jrow = [jnp.full((L,), j, jnp.int32) for j in range(8)]

        pltpu.sync_copy(zeros_hbm, acc)

        def issue(t, b):
            row = c * NCHUNK + t
            eb = row * CH
            pltpu.async_copy(il_hbm.at[row], il_v.at[b], isem.at[b])
            pltpu.async_copy(
                edge_hbm.at[pl.ds(s * 8, 8), pl.ds(eb, CH)],
                stage.at[b], rsem.at[b])

        def wait(b):
            pltpu.make_async_copy(il_hbm.at[0], il_v.at[b], isem.at[b]).wait()
            pltpu.make_async_copy(
                edge_hbm.at[pl.ds(0, 8), pl.ds(0, CH)],
                stage.at[b], rsem.at[b]).wait()

        def consume(b):
            wait(b)

            def groups(gb, carry):
                for u in range(GU):
                    off = gb * (GU * L) + u * L
                    idx16 = il_v[b, 0, pl.ds(off, L)]
                    v_last = plsc.bitcast(il_v[b, 1, pl.ds(off, L)],
                                          jnp.float32)
                    v_row0 = stage[b, 0, pl.ds(off, L)]
                    v0 = jnp.where(is_tile0, v_last, v_row0)
                    plsc.addupdate_scatter(acc, [jrow[0], idx16], v0)
                    for j in range(1, 8):
                        vals = stage[b, j, pl.ds(off, L)]
                        plsc.addupdate_scatter(acc, [jrow[j], idx16], vals)
                return carry

            lax.fori_loop(0, GROUPS // GU, groups, 0)

        for b in range(NB):
            issue(b, b)

        def body(r, carry):
            for b in range(NB):
                t = r * NB + b
                consume(b)
                nt = t + NB

                @pl.when(nt < NCHUNK)
                def _():
                    issue(nt, b)
            return carry

        lax.fori_loop(0, NCHUNK // NB, body, 0)
        for t in range(NB * (NCHUNK // NB), NCHUNK):
            consume(t % NB)

        pltpu.sync_copy(acc, part_hbm.at[c, pl.ds(s * 8, 8)])

    return k(edge_t, il3, zeros8)


def _combine(vertex_attr, partial):
    def body(v_ref, p_ref, o_ref):
        p = (p_ref[0] + p_ref[1]).T
        o_ref[:, :D] = v_ref[...]
        o_ref[:, D:] = jnp.concatenate([p[:, 1:], p[:, :1]], axis=1)

    return pl.pallas_call(
        body,
        grid=(10,),
        in_specs=[
            pl.BlockSpec((1024, D), lambda i: (i, 0)),
            pl.BlockSpec((NC, D, 1024), lambda i: (0, 0, i)),
        ],
        out_specs=pl.BlockSpec((1024, 2 * D), lambda i: (i, 0)),
        out_shape=jax.ShapeDtypeStruct((N, 2 * D), jnp.float32),
    )(vertex_attr, partial)


def kernel(vertex_attr, edgeij_pair, edge_attr, g, batch):
    edge_t = edge_attr.T
    dst2 = edgeij_pair[1].reshape(ROWS, 1, CH)
    last2 = lax.bitcast_convert_type(
        edge_attr[:, D].reshape(ROWS, 1, CH), jnp.int32)
    il3 = jnp.concatenate([dst2, last2], axis=1)
    zeros8 = jnp.zeros((8, N_PAD), dtype=jnp.float32)
    partial = _sc_partial(edge_t, il3, zeros8)
    return _combine(vertex_attr, partial)


# speedup vs baseline: 1.0160x; 1.0160x over previous
import functools

import jax
import jax.numpy as jnp
from jax import lax
from jax.experimental import pallas as pl
from jax.experimental.pallas import tpu as pltpu
from jax.experimental.pallas import tpu_sc as plsc

N = 10000
E = 320000
D = 128

NC = 2
NS = 16
L = 16

E2 = E // NC
CH = 1280
NCHUNK = E2 // CH
ROWS = E // CH
GROUPS = CH // L
GU = 16
NB = 2

N_PAD = 10240


def _sc_partial(edge_t, il3, zeros8):
    mesh = plsc.VectorSubcoreMesh(core_axis_name="c", subcore_axis_name="s")

    @functools.partial(
        pl.kernel,
        out_type=jax.ShapeDtypeStruct((NC, D, N_PAD), jnp.float32),
        mesh=mesh,
        scratch_types=[
            pltpu.VMEM((NB, 2, CH), jnp.int32),
            pltpu.VMEM((NB, 8, CH), jnp.float32),
            pltpu.VMEM((8, N_PAD), jnp.float32),
            pltpu.SemaphoreType.DMA((NB,)),
            pltpu.SemaphoreType.DMA((NB,)),
        ],
        compiler_params=pltpu.CompilerParams(needs_layout_passes=False),
    )
    def k(edge_hbm, il_hbm, zeros_hbm, part_hbm,
          il_v, stage, acc, isem, rsem):
        c = lax.axis_index("c")
        s = lax.axis_index("s")
        is_tile0 = s == 0
        jrow = [jnp.full((L,), j, jnp.int32) for j in range(8)]

        pltpu.sync_copy(zeros_hbm, acc)

        def issue(t, b):
            row = c * NCHUNK + t
            eb = row * CH
            pltpu.async_copy(il_hbm.at[row], il_v.at[b], isem.at[b])
            pltpu.async_copy(
                edge_hbm.at[pl.ds(s * 8, 8), pl.ds(eb, CH)],
                stage.at[b], rsem.at[b])

        def wait(b):
            pltpu.make_async_copy(il_hbm.at[0], il_v.at[b], isem.at[b]).wait()
            pltpu.make_async_copy(
                edge_hbm.at[pl.ds(0, 8), pl.ds(0, CH)],
                stage.at[b], rsem.at[b]).wait()

        def consume(b):
            wait(b)

            def groups(gb, carry):
                for u in range(GU):
                    off = gb * (GU * L) + u * L
                    idx16 = il_v[b, 0, pl.ds(off, L)]
                    v_last = plsc.bitcast(il_v[b, 1, pl.ds(off, L)],
                                          jnp.float32)
                    v_row0 = stage[b, 0, pl.ds(off, L)]
                    v0 = jnp.where(is_tile0, v_last, v_row0)
                    plsc.addupdate_scatter(acc, [jrow[0], idx16], v0)
                    for j in range(1, 8):
                        vals = stage[b, j, pl.ds(off, L)]
                        plsc.addupdate_scatter(acc, [jrow[j], idx16], vals)
                return carry

            lax.fori_loop(0, GROUPS // GU, groups, 0)

        for b in range(NB):
            issue(b, b)

        def body(r, carry):
            for b in range(NB):
                t = r * NB + b
                consume(b)
                nt = t + NB

                @pl.when(nt < NCHUNK)
                def _():
                    issue(nt, b)
            return carry

        lax.fori_loop(0, NCHUNK // NB, body, 0)
        for t in range(NB * (NCHUNK // NB), NCHUNK):
            consume(t % NB)

        pltpu.sync_copy(acc, part_hbm.at[c, pl.ds(s * 8, 8)])

    return k(edge_t, il3, zeros8)


def _combine(vertex_attr, partial):
    def body(v_ref, p_ref, o_ref):
        p = (p_ref[0] + p_ref[1]).T
        o_ref[:, :D] = v_ref[...]
        o_ref[:, D:] = jnp.concatenate([p[:, 1:], p[:, :1]], axis=1)

    return pl.pallas_call(
        body,
        grid=(10,),
        in_specs=[
            pl.BlockSpec((1024, D), lambda i: (i, 0)),
            pl.BlockSpec((NC, D, 1024), lambda i: (0, 0, i)),
        ],
        out_specs=pl.BlockSpec((1024, 2 * D), lambda i: (i, 0)),
        out_shape=jax.ShapeDtypeStruct((N, 2 * D), jnp.float32),
    )(vertex_attr, partial)


def kernel(vertex_attr, edgeij_pair, edge_attr, g, batch):
    edge_t = edge_attr.T
    dst2 = edgeij_pair[1].reshape(ROWS, 1, CH)
    last2 = lax.bitcast_convert_type(
        edge_attr[:, D].reshape(ROWS, 1, CH), jnp.int32)
    il3 = jnp.concatenate([dst2, last2], axis=1)
    zeros8 = jnp.zeros((8, N_PAD), dtype=jnp.float32)
    partial = _sc_partial(edge_t, il3, zeros8)
    return _combine(vertex_attr, partial)
